# baseline (speedup 1.0000x reference)
import jax
import jax.numpy as jnp
from jax.experimental import pallas as pl
from jax.experimental.pallas import tpu as pltpu
from jax.experimental.pallas import tpu_sc as plsc

EPS = 1e-12
GW = 128
BK = 8


def _sc_gather(table, idx_flat):
    n = idx_flat.shape[1]
    dim = table.shape[1]

    @pl.kernel(
        out_type=jax.ShapeDtypeStruct((n, dim), table.dtype),
        mesh=plsc.VectorSubcoreMesh(core_axis_name="core",
                                    subcore_axis_name="subcore"),
    )
    def k(tab_hbm, i_hbm, o_hbm):
        def body(i_vmem, o_vmem):
            pltpu.sync_copy(tab_hbm.at[i_vmem.at[0]], o_vmem)

        pltpu.emit_pipeline(
            body,
            grid=(n // GW,),
            in_specs=[pl.BlockSpec((1, GW), index_map=lambda i: (0, i))],
            out_specs=[pl.BlockSpec((GW, dim), index_map=lambda i: (i, 0))],
            core_axis_name=("core", "subcore"),
            dimension_semantics=(pltpu.PARALLEL,),
        )(i_hbm, o_hbm)

    return k(table, idx_flat)


def _tc_layernorm(x, pos_tile, gamma, beta):
    n, dim = x.shape
    rows = pos_tile.shape[0]

    def body(x_ref, pos_ref, g_ref, bt_ref, o_ref):
        v = x_ref[...] + pos_ref[...]
        mean = jnp.mean(v, axis=-1, keepdims=True)
        c = v - mean
        var = jnp.mean(c * c, axis=-1, keepdims=True)
        o_ref[...] = c * jax.lax.rsqrt(var + EPS) * g_ref[...] + bt_ref[...]

    return pl.pallas_call(
        body,
        grid=(n // rows,),
        in_specs=[
            pl.BlockSpec((rows, dim), lambda i: (i, 0)),
            pl.BlockSpec((rows, dim), lambda i: (0, 0)),
            pl.BlockSpec((1, dim), lambda i: (0, 0)),
            pl.BlockSpec((1, dim), lambda i: (0, 0)),
        ],
        out_specs=pl.BlockSpec((rows, dim), lambda i: (i, 0)),
        out_shape=jax.ShapeDtypeStruct((n, dim), jnp.float32),
        compiler_params=pltpu.CompilerParams(
            dimension_semantics=("parallel",)),
    )(x, pos_tile, gamma, beta)


def kernel(news_batch, word_embeddings, pos_embedding, gamma, beta):
    b, sig = news_batch.shape
    vocab, dim = word_embeddings.shape
    half = dim // 2
    table2 = word_embeddings.reshape(2 * vocab, half)
    idx = news_batch.reshape(b * sig).astype(jnp.int32)
    idx2 = jnp.stack([2 * idx, 2 * idx + 1], axis=-1).reshape(1, 2 * b * sig)
    gathered = _sc_gather(table2, idx2)
    x = gathered.reshape(b * sig, dim)
    pos_tile = jnp.tile(pos_embedding.reshape(sig, dim), (BK, 1))
    y = _tc_layernorm(x, pos_tile,
                      gamma.reshape(1, dim), beta.reshape(1, dim))
    return y.reshape(b, sig, dim)


# speedup vs baseline: 1.6926x; 1.6926x over previous
import jax
import jax.numpy as jnp
from jax.experimental import pallas as pl
from jax.experimental.pallas import tpu as pltpu
from jax.experimental.pallas import tpu_sc as plsc

EPS = 1e-12
GW = 128
BK = 8


def _sc_gather(table2, idx2):
    n2 = idx2.shape[1]
    half = table2.shape[1]

    @pl.kernel(
        out_type=jax.ShapeDtypeStruct((n2, half), table2.dtype),
        mesh=plsc.VectorSubcoreMesh(core_axis_name="core",
                                    subcore_axis_name="subcore"),
    )
    def k(tab_hbm, i_hbm, o_hbm):
        def body(i_vmem, o_vmem):
            pltpu.sync_copy(tab_hbm.at[i_vmem.at[0]], o_vmem)

        pltpu.emit_pipeline(
            body,
            grid=(n2 // GW,),
            in_specs=[pl.BlockSpec((1, GW), index_map=lambda i: (0, i))],
            out_specs=[pl.BlockSpec((GW, half), index_map=lambda i: (i, 0))],
            core_axis_name=("core", "subcore"),
            dimension_semantics=(pltpu.PARALLEL,),
        )(i_hbm, o_hbm)

    return k(table2, idx2)


def _tc_layernorm(halves, pos_l, pos_r, gamma_l, gamma_r, beta_l, beta_r,
                  b, sig, dim):
    n = b * sig
    half = dim // 2
    bk = BK
    rows = bk * sig
    nblocks = n // rows

    def body(l_ref, r_ref, pl_ref, pr_ref, gl_ref, gr_ref, bl_ref, br_ref,
             o_ref):
        l = l_ref[...] + pl_ref[...]
        r = r_ref[...] + pr_ref[...]
        s = jnp.sum(l, axis=-1, keepdims=True) + jnp.sum(r, axis=-1,
                                                         keepdims=True)
        mean = s / dim
        cl = l - mean
        cr = r - mean
        var = (jnp.sum(cl * cl, axis=-1, keepdims=True)
               + jnp.sum(cr * cr, axis=-1, keepdims=True)) / dim
        inv = jax.lax.rsqrt(var + EPS)
        yl = cl * inv * gl_ref[...] + bl_ref[...]
        yr = cr * inv * gr_ref[...] + br_ref[...]
        o_ref[...] = jnp.concatenate([yl, yr], axis=-1).reshape(bk, sig, dim)

    return pl.pallas_call(
        body,
        grid=(nblocks,),
        in_specs=[
            pl.BlockSpec((rows, half), lambda i: (i, 0)),
            pl.BlockSpec((rows, half), lambda i: (i + nblocks, 0)),
            pl.BlockSpec((rows, half), lambda i: (0, 0)),
            pl.BlockSpec((rows, half), lambda i: (0, 0)),
            pl.BlockSpec((1, half), lambda i: (0, 0)),
            pl.BlockSpec((1, half), lambda i: (0, 0)),
            pl.BlockSpec((1, half), lambda i: (0, 0)),
            pl.BlockSpec((1, half), lambda i: (0, 0)),
        ],
        out_specs=pl.BlockSpec((bk, sig, dim), lambda i: (i, 0, 0)),
        out_shape=jax.ShapeDtypeStruct((b, sig, dim), jnp.float32),
    )(halves, halves, pos_l, pos_r, gamma_l, gamma_r, beta_l, beta_r)


def kernel(news_batch, word_embeddings, pos_embedding, gamma, beta):
    b, sig = news_batch.shape
    vocab, dim = word_embeddings.shape
    half = dim // 2
    table2 = word_embeddings.reshape(2 * vocab, half)
    idx = news_batch.reshape(b * sig).astype(jnp.int32)
    idx2 = jnp.concatenate([2 * idx, 2 * idx + 1]).reshape(1, 2 * b * sig)
    halves = _sc_gather(table2, idx2)
    pos2 = jnp.tile(pos_embedding.reshape(sig, dim), (BK, 1))
    return _tc_layernorm(
        halves,
        pos2[:, :half], pos2[:, half:],
        gamma[:half].reshape(1, half), gamma[half:].reshape(1, half),
        beta[:half].reshape(1, half), beta[half:].reshape(1, half),
        b, sig, dim)


# speedup vs baseline: 1.7400x; 1.0280x over previous
import jax
import jax.numpy as jnp
from jax.experimental import pallas as pl
from jax.experimental.pallas import tpu as pltpu
from jax.experimental.pallas import tpu_sc as plsc

EPS = 1e-12
GW = 128
BK = 8
CHUNKS = 2


def _sc_gather(table2, idx2):
    n2 = idx2.shape[1]
    half = table2.shape[1]

    @pl.kernel(
        out_type=jax.ShapeDtypeStruct((n2, half), table2.dtype),
        mesh=plsc.VectorSubcoreMesh(core_axis_name="core",
                                    subcore_axis_name="subcore"),
    )
    def k(tab_hbm, i_hbm, o_hbm):
        def body(i_vmem, o_vmem):
            pltpu.sync_copy(tab_hbm.at[i_vmem.at[0]], o_vmem)

        pltpu.emit_pipeline(
            body,
            grid=(n2 // GW,),
            in_specs=[pl.BlockSpec((1, GW), index_map=lambda i: (0, i))],
            out_specs=[pl.BlockSpec((GW, half), index_map=lambda i: (i, 0))],
            core_axis_name=("core", "subcore"),
            dimension_semantics=(pltpu.PARALLEL,),
        )(i_hbm, o_hbm)

    return k(table2, idx2)


def _tc_layernorm_chunk(halves, pos_l, pos_r, gamma_l, gamma_r, beta_l,
                        beta_r, carry, b_total, b_chunk, sig, dim, chunk):
    half = dim // 2
    rows = BK * sig
    nblocks = (b_chunk * sig) // rows
    off = chunk * nblocks

    def body(l_ref, r_ref, pl_ref, pr_ref, gl_ref, gr_ref, bl_ref, br_ref,
             *rest):
        o_ref = rest[-1]
        l = l_ref[...] + pl_ref[...]
        r = r_ref[...] + pr_ref[...]
        s = jnp.sum(l, axis=-1, keepdims=True) + jnp.sum(r, axis=-1,
                                                         keepdims=True)
        mean = s / dim
        cl = l - mean
        cr = r - mean
        var = (jnp.sum(cl * cl, axis=-1, keepdims=True)
               + jnp.sum(cr * cr, axis=-1, keepdims=True)) / dim
        inv = jax.lax.rsqrt(var + EPS)
        yl = cl * inv * gl_ref[...] + bl_ref[...]
        yr = cr * inv * gr_ref[...] + br_ref[...]
        o_ref[...] = jnp.concatenate([yl, yr], axis=-1).reshape(BK, sig, dim)

    in_specs = [
        pl.BlockSpec((rows, half), lambda i: (i, 0)),
        pl.BlockSpec((rows, half), lambda i: (i + nblocks, 0)),
        pl.BlockSpec((rows, half), lambda i: (0, 0)),
        pl.BlockSpec((rows, half), lambda i: (0, 0)),
        pl.BlockSpec((1, half), lambda i: (0, 0)),
        pl.BlockSpec((1, half), lambda i: (0, 0)),
        pl.BlockSpec((1, half), lambda i: (0, 0)),
        pl.BlockSpec((1, half), lambda i: (0, 0)),
    ]
    args = [halves, halves, pos_l, pos_r, gamma_l, gamma_r, beta_l, beta_r]
    aliases = {}
    if carry is not None:
        in_specs.append(pl.BlockSpec(memory_space=pltpu.MemorySpace.HBM))
        args.append(carry)
        aliases = {8: 0}

    return pl.pallas_call(
        body,
        grid=(nblocks,),
        in_specs=in_specs,
        out_specs=pl.BlockSpec((BK, sig, dim), lambda i: (off + i, 0, 0)),
        out_shape=jax.ShapeDtypeStruct((b_total, sig, dim), jnp.float32),
        input_output_aliases=aliases,
    )(*args)


def kernel(news_batch, word_embeddings, pos_embedding, gamma, beta):
    b, sig = news_batch.shape
    vocab, dim = word_embeddings.shape
    half = dim // 2
    table2 = word_embeddings.reshape(2 * vocab, half)
    pos2 = jnp.tile(pos_embedding.reshape(sig, dim), (BK, 1))
    pos_l, pos_r = pos2[:, :half], pos2[:, half:]
    g_l = gamma[:half].reshape(1, half)
    g_r = gamma[half:].reshape(1, half)
    b_l = beta[:half].reshape(1, half)
    b_r = beta[half:].reshape(1, half)

    bc = b // CHUNKS
    gathers = []
    for c in range(CHUNKS):
        idx = news_batch[c * bc:(c + 1) * bc].reshape(bc * sig)
        idx = idx.astype(jnp.int32)
        idx2 = jnp.concatenate([2 * idx, 2 * idx + 1]).reshape(1, 2 * bc * sig)
        gathers.append(_sc_gather(table2, idx2))

    out = None
    for c in range(CHUNKS):
        out = _tc_layernorm_chunk(gathers[c], pos_l, pos_r, g_l, g_r,
                                  b_l, b_r, out, b, bc, sig, dim, c)
    return out


# speedup vs baseline: 2.1023x; 1.2082x over previous
import jax
import jax.numpy as jnp
from jax.experimental import pallas as pl
from jax.experimental.pallas import tpu as pltpu
from jax.experimental.pallas import tpu_sc as plsc

EPS = 1e-12
GR = 64
BK = 8
CHUNKS = 1


def _sc_gather(table, idx):
    m = idx.shape[1]
    dim = table.shape[1]

    @pl.kernel(
        out_type=jax.ShapeDtypeStruct((m, dim), table.dtype),
        mesh=plsc.VectorSubcoreMesh(core_axis_name="core",
                                    subcore_axis_name="subcore"),
    )
    def k(tab_hbm, i_hbm, o_hbm):
        def body(indices, i_vmem, o_vmem):
            (step,) = indices
            base = (step % 2) * GR
            pltpu.sync_copy(tab_hbm.at[i_vmem.at[0, pl.ds(base, GR)]], o_vmem)

        pltpu.emit_pipeline(
            body,
            grid=(m // GR,),
            in_specs=[pl.BlockSpec((1, 2 * GR), index_map=lambda i: (0, i // 2))],
            out_specs=[pl.BlockSpec((GR, dim), index_map=lambda i: (i, 0))],
            core_axis_name=("core", "subcore"),
            dimension_semantics=(pltpu.PARALLEL,),
            _explicit_indices=True,
        )(i_hbm, o_hbm)

    return k(table, idx)


def _tc_layernorm_chunk(x, pos, gamma, beta, carry, b_total, b_chunk, sig,
                        dim, chunk):
    rows = BK * sig
    nblocks = (b_chunk * sig) // rows
    off = chunk * nblocks

    def body(x_ref, p_ref, g_ref, bt_ref, *rest):
        o_ref = rest[-1]
        v = x_ref[...] + p_ref[...]
        mean = jnp.mean(v, axis=-1, keepdims=True)
        c = v - mean
        var = jnp.mean(c * c, axis=-1, keepdims=True)
        y = c * jax.lax.rsqrt(var + EPS) * g_ref[...] + bt_ref[...]
        o_ref[...] = y.reshape(BK, sig, dim)

    in_specs = [
        pl.BlockSpec((rows, dim), lambda i: (i, 0)),
        pl.BlockSpec((rows, dim), lambda i: (0, 0)),
        pl.BlockSpec((1, dim), lambda i: (0, 0)),
        pl.BlockSpec((1, dim), lambda i: (0, 0)),
    ]
    args = [x, pos, gamma, beta]
    aliases = {}
    if carry is not None:
        in_specs.append(pl.BlockSpec(memory_space=pltpu.MemorySpace.HBM))
        args.append(carry)
        aliases = {4: 0}

    return pl.pallas_call(
        body,
        grid=(nblocks,),
        in_specs=in_specs,
        out_specs=pl.BlockSpec((BK, sig, dim), lambda i: (off + i, 0, 0)),
        out_shape=jax.ShapeDtypeStruct((b_total, sig, dim), jnp.float32),
        input_output_aliases=aliases,
    )(*args)


def kernel(news_batch, word_embeddings, pos_embedding, gamma, beta):
    b, sig = news_batch.shape
    vocab, dim = word_embeddings.shape
    pos_tile = jnp.tile(pos_embedding.reshape(sig, dim), (BK, 1))
    g2 = gamma.reshape(1, dim)
    b2 = beta.reshape(1, dim)

    bc = b // CHUNKS
    gathers = []
    for c in range(CHUNKS):
        idx = news_batch[c * bc:(c + 1) * bc].reshape(1, bc * sig)
        gathers.append(_sc_gather(word_embeddings, idx.astype(jnp.int32)))

    out = None
    for c in range(CHUNKS):
        out = _tc_layernorm_chunk(gathers[c], pos_tile, g2, b2, out,
                                  b, bc, sig, dim, c)
    return out


# speedup vs baseline: 2.1816x; 1.0377x over previous
import jax
import jax.numpy as jnp
from jax.experimental import pallas as pl
from jax.experimental.pallas import tpu as pltpu
from jax.experimental.pallas import tpu_sc as plsc

EPS = 1e-12
GR = 64
BK = 8
CHUNKS = 2


def _sc_gather(table, idx):
    m = idx.shape[1]
    dim = table.shape[1]

    @pl.kernel(
        out_type=jax.ShapeDtypeStruct((m, dim), table.dtype),
        mesh=plsc.VectorSubcoreMesh(core_axis_name="core",
                                    subcore_axis_name="subcore"),
    )
    def k(tab_hbm, i_hbm, o_hbm):
        def body(indices, i_vmem, o_vmem):
            (step,) = indices
            base = (step % 2) * GR
            pltpu.sync_copy(tab_hbm.at[i_vmem.at[0, pl.ds(base, GR)]], o_vmem)

        pltpu.emit_pipeline(
            body,
            grid=(m // GR,),
            in_specs=[pl.BlockSpec((1, 2 * GR), index_map=lambda i: (0, i // 2))],
            out_specs=[pl.BlockSpec((GR, dim), index_map=lambda i: (i, 0))],
            core_axis_name=("core", "subcore"),
            dimension_semantics=(pltpu.PARALLEL,),
            _explicit_indices=True,
        )(i_hbm, o_hbm)

    return k(table, idx)


def _tc_layernorm_chunk(x, pos, gamma, beta, carry, b_total, b_chunk, sig,
                        dim, chunk):
    rows = BK * sig
    nblocks = (b_chunk * sig) // rows
    off = chunk * nblocks

    def body(x_ref, p_ref, g_ref, bt_ref, *rest):
        o_ref = rest[-1]
        v = x_ref[...] + p_ref[...]
        mean = jnp.mean(v, axis=-1, keepdims=True)
        c = v - mean
        var = jnp.mean(c * c, axis=-1, keepdims=True)
        y = c * jax.lax.rsqrt(var + EPS) * g_ref[...] + bt_ref[...]
        o_ref[...] = y.reshape(BK, sig, dim)

    in_specs = [
        pl.BlockSpec((rows, dim), lambda i: (i, 0)),
        pl.BlockSpec((rows, dim), lambda i: (0, 0)),
        pl.BlockSpec((1, dim), lambda i: (0, 0)),
        pl.BlockSpec((1, dim), lambda i: (0, 0)),
    ]
    args = [x, pos, gamma, beta]
    aliases = {}
    if carry is not None:
        in_specs.append(pl.BlockSpec(memory_space=pltpu.MemorySpace.HBM))
        args.append(carry)
        aliases = {4: 0}

    return pl.pallas_call(
        body,
        grid=(nblocks,),
        in_specs=in_specs,
        out_specs=pl.BlockSpec((BK, sig, dim), lambda i: (off + i, 0, 0)),
        out_shape=jax.ShapeDtypeStruct((b_total, sig, dim), jnp.float32),
        input_output_aliases=aliases,
    )(*args)


def kernel(news_batch, word_embeddings, pos_embedding, gamma, beta):
    b, sig = news_batch.shape
    vocab, dim = word_embeddings.shape
    pos_tile = jnp.tile(pos_embedding.reshape(sig, dim), (BK, 1))
    g2 = gamma.reshape(1, dim)
    b2 = beta.reshape(1, dim)

    bc = b // CHUNKS
    gathers = []
    for c in range(CHUNKS):
        idx = news_batch[c * bc:(c + 1) * bc].reshape(1, bc * sig)
        gathers.append(_sc_gather(word_embeddings, idx.astype(jnp.int32)))

    out = None
    for c in range(CHUNKS):
        out = _tc_layernorm_chunk(gathers[c], pos_tile, g2, b2, out,
                                  b, bc, sig, dim, c)
    return out


# speedup vs baseline: 3.3034x; 1.5142x over previous
import jax
import jax.numpy as jnp
from jax.experimental import pallas as pl
from jax.experimental.pallas import tpu as pltpu
from jax.experimental.pallas import tpu_sc as plsc

EPS = 1e-12
GR = 64
ROWS = 512
CHUNKS = 2


def _sc_gather(table, idx):
    m = idx.shape[1]
    dim = table.shape[1]

    @pl.kernel(
        out_type=jax.ShapeDtypeStruct((m, dim), table.dtype),
        mesh=plsc.VectorSubcoreMesh(core_axis_name="core",
                                    subcore_axis_name="subcore"),
    )
    def k(tab_hbm, i_hbm, o_hbm):
        def body(indices, i_vmem, o_vmem):
            (step,) = indices
            base = (step % 2) * GR
            pltpu.sync_copy(tab_hbm.at[i_vmem.at[0, pl.ds(base, GR)]], o_vmem)

        pltpu.emit_pipeline(
            body,
            grid=(m // GR,),
            in_specs=[pl.BlockSpec((1, 2 * GR), index_map=lambda i: (0, i // 2))],
            out_specs=[pl.BlockSpec((GR, dim), index_map=lambda i: (i, 0))],
            core_axis_name=("core", "subcore"),
            dimension_semantics=(pltpu.PARALLEL,),
            _explicit_indices=True,
        )(i_hbm, o_hbm)

    return k(table, idx)


def _tc_layernorm_chunk(x, pos, gamma, beta, carry, b, sig, sig_chunk, dim,
                        chunk):
    nblocks = (sig_chunk * b) // ROWS
    per_band = b // ROWS
    s_off = chunk * sig_chunk

    def body(x_ref, p_ref, g_ref, bt_ref, *rest):
        o_ref = rest[-1]
        v = x_ref[...] + p_ref[0]
        mean = jnp.mean(v, axis=-1, keepdims=True)
        c = v - mean
        var = jnp.mean(c * c, axis=-1, keepdims=True)
        y = c * jax.lax.rsqrt(var + EPS) * g_ref[...] + bt_ref[...]
        o_ref[...] = y.reshape(1, ROWS, dim)

    in_specs = [
        pl.BlockSpec((ROWS, dim), lambda i: (i, 0)),
        pl.BlockSpec((1, 1, dim), lambda i: (s_off + i // per_band, 0, 0)),
        pl.BlockSpec((1, dim), lambda i: (0, 0)),
        pl.BlockSpec((1, dim), lambda i: (0, 0)),
    ]
    args = [x, pos, gamma, beta]
    aliases = {}
    if carry is not None:
        in_specs.append(pl.BlockSpec(memory_space=pltpu.MemorySpace.HBM))
        args.append(carry)
        aliases = {4: 0}

    return pl.pallas_call(
        body,
        grid=(nblocks,),
        in_specs=in_specs,
        out_specs=pl.BlockSpec(
            (1, ROWS, dim),
            lambda i: (s_off + i // per_band, i % per_band, 0)),
        out_shape=jax.ShapeDtypeStruct((sig, b, dim), jnp.float32),
        input_output_aliases=aliases,
    )(*args)


def kernel(news_batch, word_embeddings, pos_embedding, gamma, beta):
    b, sig = news_batch.shape
    vocab, dim = word_embeddings.shape
    pos2 = pos_embedding.reshape(sig, 1, dim)
    g2 = gamma.reshape(1, dim)
    b2 = beta.reshape(1, dim)

    idx_t = news_batch.T.astype(jnp.int32)
    sc = sig // CHUNKS
    gathers = []
    for c in range(CHUNKS):
        idx = idx_t[c * sc:(c + 1) * sc].reshape(1, sc * b)
        gathers.append(_sc_gather(word_embeddings, idx))

    out = None
    for c in range(CHUNKS):
        out = _tc_layernorm_chunk(gathers[c], pos2, g2, b2, out,
                                  b, sig, sc, dim, c)
    return out.transpose(1, 0, 2)


# speedup vs baseline: 3.3888x; 1.0258x over previous
import jax
import jax.numpy as jnp
from jax.experimental import pallas as pl
from jax.experimental.pallas import tpu as pltpu
from jax.experimental.pallas import tpu_sc as plsc

EPS = 1e-12
GR = 64
ROWS = 512
CHUNKS = 5


def _sc_gather(table, idx):
    m = idx.shape[1]
    dim = table.shape[1]

    @pl.kernel(
        out_type=jax.ShapeDtypeStruct((m, dim), table.dtype),
        mesh=plsc.VectorSubcoreMesh(core_axis_name="core",
                                    subcore_axis_name="subcore"),
    )
    def k(tab_hbm, i_hbm, o_hbm):
        def body(indices, i_vmem, o_vmem):
            (step,) = indices
            base = (step % 2) * GR
            pltpu.sync_copy(tab_hbm.at[i_vmem.at[0, pl.ds(base, GR)]], o_vmem)

        pltpu.emit_pipeline(
            body,
            grid=(m // GR,),
            in_specs=[pl.BlockSpec((1, 2 * GR), index_map=lambda i: (0, i // 2))],
            out_specs=[pl.BlockSpec((GR, dim), index_map=lambda i: (i, 0))],
            core_axis_name=("core", "subcore"),
            dimension_semantics=(pltpu.PARALLEL,),
            _explicit_indices=True,
        )(i_hbm, o_hbm)

    return k(table, idx)


def _tc_layernorm_chunk(x, pos, gamma, beta, carry, b, sig, sig_chunk, dim,
                        chunk):
    nblocks = (sig_chunk * b) // ROWS
    per_band = b // ROWS
    s_off = chunk * sig_chunk

    def body(x_ref, p_ref, g_ref, bt_ref, *rest):
        o_ref = rest[-1]
        v = x_ref[...] + p_ref[0]
        mean = jnp.mean(v, axis=-1, keepdims=True)
        c = v - mean
        var = jnp.mean(c * c, axis=-1, keepdims=True)
        y = c * jax.lax.rsqrt(var + EPS) * g_ref[...] + bt_ref[...]
        o_ref[...] = y.reshape(1, ROWS, dim)

    in_specs = [
        pl.BlockSpec((ROWS, dim), lambda i: (i, 0)),
        pl.BlockSpec((1, 1, dim), lambda i: (s_off + i // per_band, 0, 0)),
        pl.BlockSpec((1, dim), lambda i: (0, 0)),
        pl.BlockSpec((1, dim), lambda i: (0, 0)),
    ]
    args = [x, pos, gamma, beta]
    aliases = {}
    if carry is not None:
        in_specs.append(pl.BlockSpec(memory_space=pltpu.MemorySpace.HBM))
        args.append(carry)
        aliases = {4: 0}

    return pl.pallas_call(
        body,
        grid=(nblocks,),
        in_specs=in_specs,
        out_specs=pl.BlockSpec(
            (1, ROWS, dim),
            lambda i: (s_off + i // per_band, i % per_band, 0)),
        out_shape=jax.ShapeDtypeStruct((sig, b, dim), jnp.float32),
        input_output_aliases=aliases,
    )(*args)


def kernel(news_batch, word_embeddings, pos_embedding, gamma, beta):
    b, sig = news_batch.shape
    vocab, dim = word_embeddings.shape
    pos2 = pos_embedding.reshape(sig, 1, dim)
    g2 = gamma.reshape(1, dim)
    b2 = beta.reshape(1, dim)

    idx_t = news_batch.T.astype(jnp.int32)
    sc = sig // CHUNKS
    gathers = []
    for c in range(CHUNKS):
        idx = idx_t[c * sc:(c + 1) * sc].reshape(1, sc * b)
        gathers.append(_sc_gather(word_embeddings, idx))

    out = None
    for c in range(CHUNKS):
        out = _tc_layernorm_chunk(gathers[c], pos2, g2, b2, out,
                                  b, sig, sc, dim, c)
    return out.transpose(1, 0, 2)
